# baseline (speedup 1.0000x reference)
import functools
import math

import jax
import jax.numpy as jnp
from jax import lax
from jax.experimental import pallas as pl
from jax.experimental.pallas import tpu as pltpu
from jax.experimental.pallas import tpu_sc as plsc

_SQRT_2_OVER_PI = math.sqrt(2.0 / math.pi)
_A = -2.0 * _SQRT_2_OVER_PI
_B = _A * 0.044715

_N = 4 * 8192 * 2048
_NW = 32
_PER_W = _N // _NW
_CHUNK = 16384
_NCHUNK = _PER_W // _CHUNK
_L = 16


def _sc_body(x_hbm, o_hbm, xv, ov):
    c = lax.axis_index("c")
    s = lax.axis_index("s")
    wid = s * 2 + c
    base = wid * _PER_W

    def chunk_body(j, carry):
        off = base + j * _CHUNK
        pltpu.sync_copy(x_hbm.at[pl.ds(off, _CHUNK)], xv)

        def vec_body(i, carry2):
            xx = xv[pl.ds(i * _L, _L)]
            x2 = xx * xx
            u = _A * xx + _B * (x2 * xx)
            ov[pl.ds(i * _L, _L)] = xx / (1.0 + jnp.exp(u))
            return carry2

        lax.fori_loop(0, _CHUNK // _L, vec_body, 0)
        pltpu.sync_copy(ov, o_hbm.at[pl.ds(off, _CHUNK)])
        return carry

    lax.fori_loop(0, _NCHUNK, chunk_body, 0)


_sc_gelu = functools.partial(
    pl.kernel,
    mesh=plsc.VectorSubcoreMesh(core_axis_name="c", subcore_axis_name="s"),
    out_type=jax.ShapeDtypeStruct((_N,), jnp.float32),
    scratch_types=[
        pltpu.VMEM((_CHUNK,), jnp.float32),
        pltpu.VMEM((_CHUNK,), jnp.float32),
    ],
)(_sc_body)


def kernel(x, logit_decay, log_tau, log_blend):
    del logit_decay, log_tau, log_blend
    out = _sc_gelu(x.reshape(_N))
    return out.reshape(x.shape)


# speedup vs baseline: 7.4741x; 7.4741x over previous
import math

import jax
import jax.numpy as jnp
from jax.experimental import pallas as pl
from jax.experimental.pallas import tpu as pltpu

_SQRT_2_OVER_PI = math.sqrt(2.0 / math.pi)

_ROWS = 32768
_COLS = 2048
_BLOCK_ROWS = 1024


def _gelu_block(x_ref, o_ref):
    x = x_ref[...]
    u = _SQRT_2_OVER_PI * (x + 0.044715 * (x * x * x))
    o_ref[...] = 0.5 * x * (1.0 + jnp.tanh(u))


def kernel(x, logit_decay, log_tau, log_blend):
    del logit_decay, log_tau, log_blend
    x2 = x.reshape(_ROWS, _COLS)
    out = pl.pallas_call(
        _gelu_block,
        grid=(_ROWS // _BLOCK_ROWS,),
        in_specs=[pl.BlockSpec((_BLOCK_ROWS, _COLS), lambda i: (i, 0))],
        out_specs=pl.BlockSpec((_BLOCK_ROWS, _COLS), lambda i: (i, 0)),
        out_shape=jax.ShapeDtypeStruct((_ROWS, _COLS), x.dtype),
        compiler_params=pltpu.CompilerParams(vmem_limit_bytes=100 * 1024 * 1024),
    )(x2)
    return out.reshape(x.shape)
